# baseline (speedup 1.0000x reference)
import jax
import jax.numpy as jnp
from jax.experimental import pallas as pl
from jax.experimental.pallas import tpu as pltpu

_DIM = 2048
_INTER = 1408
_E = 16
_NS = 2
_GE = _E + _NS
_TILE = 128
_J = _INTER // _TILE


def _moe_body(x_ref, gate_ref, w1_ref, ws1_ref, w2_ref, ws2_ref,
              b1_ref, b2_ref, out_ref, wi_ref):
    e = pl.program_id(0)
    j = pl.program_id(1)

    @pl.when(jnp.logical_and(e == 0, j == 0))
    def _init():
        logits = jax.lax.dot_general(
            x_ref[...], gate_ref[...], (((1,), (1,)), ((), ())),
            preferred_element_type=jnp.float32)
        m = jnp.max(logits, axis=1, keepdims=True)
        p = jnp.exp(logits - m)
        scores = p / jnp.sum(p, axis=1, keepdims=True)
        ii = jax.lax.broadcasted_iota(jnp.int32, scores.shape, 1)
        m1 = jnp.max(scores, axis=1, keepdims=True)
        a1 = jnp.min(jnp.where(scores == m1, ii, _E), axis=1, keepdims=True)
        oh1 = ii == a1
        s2 = jnp.where(oh1, -1.0, scores)
        m2 = jnp.max(s2, axis=1, keepdims=True)
        a2 = jnp.min(jnp.where(s2 == m2, ii, _E), axis=1, keepdims=True)
        wi_ref[...] = jnp.where(oh1 | (ii == a2), scores, 0.0)
        out_ref[...] = jnp.zeros_like(out_ref)

    wi = wi_ref[...]
    ii = jax.lax.broadcasted_iota(jnp.int32, wi.shape, 1)
    wcol = jnp.sum(jnp.where(ii == e, wi, 0.0), axis=1, keepdims=True)
    we = jnp.where(e < _E, wcol, 1.0)

    @pl.when(j == 0)
    def _bias2():
        out_ref[...] += we * b2_ref[0]

    def ffn_tile(w1blk, w2blk):
        h = jax.lax.dot_general(
            x_ref[...], w1blk, (((1,), (1,)), ((), ())),
            preferred_element_type=jnp.float32)
        h = jnp.maximum(h + b1_ref[0], 0.0) * we
        out_ref[...] += jax.lax.dot_general(
            h, w2blk, (((1,), (1,)), ((), ())),
            preferred_element_type=jnp.float32)

    @pl.when(e < _E)
    def _routed():
        ffn_tile(w1_ref[...], w2_ref[...])

    @pl.when(e >= _E)
    def _shared():
        ffn_tile(ws1_ref[...], ws2_ref[...])


def kernel(x, gate_w, W1, b1, W2, b2, Ws1, bs1, Ws2, bs2):
    orig_shape = x.shape
    xt = x.reshape(-1, _DIM)
    T = xt.shape[0]
    b1cat = jnp.concatenate(
        [b1, bs1.reshape(_NS, _INTER)], axis=0).reshape(_GE, 1, _INTER)
    b2cat = jnp.concatenate(
        [b2, bs2[None, :], jnp.zeros((1, _DIM), b2.dtype)],
        axis=0).reshape(_GE, 1, _DIM)

    jlast = _J - 1
    out = pl.pallas_call(
        _moe_body,
        grid=(_GE, _J),
        in_specs=[
            pl.BlockSpec((T, _DIM), lambda e, j: (0, 0)),
            pl.BlockSpec((_E, _DIM), lambda e, j: (0, 0)),
            pl.BlockSpec((None, _TILE, _DIM),
                         lambda e, j: (jnp.minimum(e, _E - 1),
                                       jnp.where(e < _E, j, jlast), 0)),
            pl.BlockSpec((_TILE, _DIM),
                         lambda e, j: (jnp.where(e < _E, 0,
                                                 (e - _E) * _J + j), 0)),
            pl.BlockSpec((None, _DIM, _TILE),
                         lambda e, j: (jnp.minimum(e, _E - 1), 0,
                                       jnp.where(e < _E, j, jlast))),
            pl.BlockSpec((_DIM, _TILE),
                         lambda e, j: (0, jnp.where(e < _E, 0,
                                                    (e - _E) * _J + j))),
            pl.BlockSpec((None, 1, _TILE), lambda e, j: (e, 0, j)),
            pl.BlockSpec((None, 1, _DIM), lambda e, j: (e, 0, 0)),
        ],
        out_specs=pl.BlockSpec((T, _DIM), lambda e, j: (0, 0)),
        out_shape=jax.ShapeDtypeStruct((T, _DIM), jnp.float32),
        scratch_shapes=[pltpu.VMEM((T, _E), jnp.float32)],
        compiler_params=pltpu.CompilerParams(
            dimension_semantics=("arbitrary", "arbitrary")),
    )(xt, gate_w, W1, Ws1, W2, Ws2, b1cat, b2cat)
    return out.reshape(orig_shape)
